# baseline (device time: 14930 ns/iter reference)
import jax
import jax.numpy as jnp
from jax import lax
from jax.experimental import pallas as pl
from jax.experimental.pallas import tpu as pltpu

N_DEV = 4
NUM_CHUNKS = 4
N_PEERS = 3


def kernel(x, W1, W2):
    m, _ = x.shape
    n = W2.shape[1]
    mc = m // NUM_CHUNKS

    def body(x_ref, w1_ref, w2_ref, out_ref, p_ref, comm_ref,
             send_sems, recv_sems):
        my_pos = lax.axis_index("i")
        peers = [my_pos ^ 1, 3 - my_pos, my_pos ^ 2]

        barrier_sem = pltpu.get_barrier_semaphore()
        for nbr in peers:
            pl.semaphore_signal(
                barrier_sem, inc=1,
                device_id=(nbr,), device_id_type=pl.DeviceIdType.MESH,
            )
        pl.semaphore_wait(barrier_sem, N_PEERS)

        def push(k, chunk):
            slot = k * NUM_CHUNKS + chunk
            return pltpu.make_async_remote_copy(
                src_ref=p_ref.at[pl.ds(chunk * mc, mc), :],
                dst_ref=comm_ref.at[slot],
                send_sem=send_sems.at[slot],
                recv_sem=recv_sems.at[slot],
                device_id=(peers[k],),
                device_id_type=pl.DeviceIdType.MESH,
            )

        rdmas = {}
        for c in range(NUM_CHUNKS):
            rows = pl.ds(c * mc, mc)
            hidden = jnp.maximum(
                jnp.dot(x_ref[rows, :], w1_ref[:, :],
                        preferred_element_type=jnp.float32),
                0.0,
            )
            p_ref[rows, :] = jnp.dot(hidden, w2_ref[:, :],
                                     preferred_element_type=jnp.float32)
            for k in range(N_PEERS):
                rdmas[(k, c)] = push(k, c)
                rdmas[(k, c)].start()

        for c in range(NUM_CHUNKS):
            rows = pl.ds(c * mc, mc)
            for k in range(N_PEERS):
                rdmas[(k, c)].wait_recv()
            out_ref[rows, :] = (
                p_ref[rows, :]
                + comm_ref[0 * NUM_CHUNKS + c, :, :]
                + comm_ref[1 * NUM_CHUNKS + c, :, :]
                + comm_ref[2 * NUM_CHUNKS + c, :, :]
            )

        for c in range(NUM_CHUNKS):
            for k in range(N_PEERS):
                rdmas[(k, c)].wait_send()

    return pl.pallas_call(
        body,
        out_shape=jax.ShapeDtypeStruct((m, n), jnp.float32),
        in_specs=[
            pl.BlockSpec(memory_space=pltpu.VMEM),
            pl.BlockSpec(memory_space=pltpu.VMEM),
            pl.BlockSpec(memory_space=pltpu.VMEM),
        ],
        out_specs=pl.BlockSpec(memory_space=pltpu.VMEM),
        scratch_shapes=[
            pltpu.VMEM((m, n), jnp.float32),
            pltpu.VMEM((N_PEERS * NUM_CHUNKS, mc, n), jnp.float32),
            pltpu.SemaphoreType.DMA((N_PEERS * NUM_CHUNKS,)),
            pltpu.SemaphoreType.DMA((N_PEERS * NUM_CHUNKS,)),
        ],
        compiler_params=pltpu.CompilerParams(collective_id=0),
    )(x, W1, W2)


# device time: 13954 ns/iter; 1.0699x vs baseline; 1.0699x over previous
import jax
import jax.numpy as jnp
from jax import lax
from jax.experimental import pallas as pl
from jax.experimental.pallas import tpu as pltpu

N_DEV = 4
NUM_CHUNKS = 4


def kernel(x, W1, W2):
    m, _ = x.shape
    n = W2.shape[1]
    mc = m // NUM_CHUNKS

    def body(x_ref, w1_ref, w2_ref, out_ref, comm_ref, send_sems, recv_sems):
        my_pos = lax.axis_index("i")
        peers = [my_pos ^ 1, 3 - my_pos]

        barrier_sem = pltpu.get_barrier_semaphore()
        for nbr in peers:
            pl.semaphore_signal(
                barrier_sem, inc=1,
                device_id=(nbr,), device_id_type=pl.DeviceIdType.MESH,
            )
        pl.semaphore_wait(barrier_sem, 2)

        def exchange(stage, chunk):
            slot = stage * NUM_CHUNKS + chunk
            return pltpu.make_async_remote_copy(
                src_ref=out_ref.at[pl.ds(chunk * mc, mc), :],
                dst_ref=comm_ref.at[slot],
                send_sem=send_sems.at[slot],
                recv_sem=recv_sems.at[slot],
                device_id=(peers[stage],),
                device_id_type=pl.DeviceIdType.MESH,
            )

        rdmas = {}
        for c in range(NUM_CHUNKS):
            rows = pl.ds(c * mc, mc)
            hidden = jnp.maximum(
                jnp.dot(x_ref[rows, :], w1_ref[:, :],
                        preferred_element_type=jnp.float32),
                0.0,
            )
            out_ref[rows, :] = jnp.dot(hidden, w2_ref[:, :],
                                       preferred_element_type=jnp.float32)
            rdmas[(0, c)] = exchange(0, c)
            rdmas[(0, c)].start()

        for c in range(NUM_CHUNKS):
            rows = pl.ds(c * mc, mc)
            rdmas[(0, c)].wait()
            out_ref[rows, :] = out_ref[rows, :] + comm_ref[c, :, :]
            rdmas[(1, c)] = exchange(1, c)
            rdmas[(1, c)].start()

        for c in range(NUM_CHUNKS):
            rows = pl.ds(c * mc, mc)
            rdmas[(1, c)].wait()
            out_ref[rows, :] = (
                out_ref[rows, :] + comm_ref[NUM_CHUNKS + c, :, :]
            )

    return pl.pallas_call(
        body,
        out_shape=jax.ShapeDtypeStruct((m, n), jnp.float32),
        in_specs=[
            pl.BlockSpec(memory_space=pltpu.VMEM),
            pl.BlockSpec(memory_space=pltpu.VMEM),
            pl.BlockSpec(memory_space=pltpu.VMEM),
        ],
        out_specs=pl.BlockSpec(memory_space=pltpu.VMEM),
        scratch_shapes=[
            pltpu.VMEM((2 * NUM_CHUNKS, mc, n), jnp.float32),
            pltpu.SemaphoreType.DMA((2 * NUM_CHUNKS,)),
            pltpu.SemaphoreType.DMA((2 * NUM_CHUNKS,)),
        ],
        compiler_params=pltpu.CompilerParams(collective_id=0),
    )(x, W1, W2)


# device time: 13618 ns/iter; 1.0963x vs baseline; 1.0247x over previous
import jax
import jax.numpy as jnp
from jax import lax
from jax.experimental import pallas as pl
from jax.experimental.pallas import tpu as pltpu

N_DEV = 4
NUM_CHUNKS = 4


def kernel(x, W1, W2):
    m, _ = x.shape
    n = W2.shape[1]
    mc = m // NUM_CHUNKS

    def body(x_ref, w1_ref, w2_ref, out_ref, comm_ref, send_sems, recv_sems):
        my_pos = lax.axis_index("i")
        peers = [my_pos ^ 1, 3 - my_pos]

        barrier_sem = pltpu.get_barrier_semaphore()
        for nbr in peers:
            pl.semaphore_signal(
                barrier_sem, inc=1,
                device_id=(nbr,), device_id_type=pl.DeviceIdType.MESH,
            )

        def exchange(stage, chunk):
            slot = stage * NUM_CHUNKS + chunk
            return pltpu.make_async_remote_copy(
                src_ref=out_ref.at[pl.ds(chunk * mc, mc), :],
                dst_ref=comm_ref.at[slot],
                send_sem=send_sems.at[slot],
                recv_sem=recv_sems.at[slot],
                device_id=(peers[stage],),
                device_id_type=pl.DeviceIdType.MESH,
            )

        rdmas = {}
        for c in range(NUM_CHUNKS):
            rows = pl.ds(c * mc, mc)
            hidden = jnp.maximum(
                jnp.dot(x_ref[rows, :], w1_ref[:, :],
                        preferred_element_type=jnp.float32),
                0.0,
            )
            out_ref[rows, :] = jnp.dot(hidden, w2_ref[:, :],
                                       preferred_element_type=jnp.float32)
            if c == 0:
                pl.semaphore_wait(barrier_sem, 2)
            rdmas[(0, c)] = exchange(0, c)
            rdmas[(0, c)].start()

        for c in range(NUM_CHUNKS):
            rows = pl.ds(c * mc, mc)
            rdmas[(0, c)].wait()
            out_ref[rows, :] = out_ref[rows, :] + comm_ref[c, :, :]
            rdmas[(1, c)] = exchange(1, c)
            rdmas[(1, c)].start()

        for c in range(NUM_CHUNKS):
            rows = pl.ds(c * mc, mc)
            rdmas[(1, c)].wait()
            out_ref[rows, :] = (
                out_ref[rows, :] + comm_ref[NUM_CHUNKS + c, :, :]
            )

    return pl.pallas_call(
        body,
        out_shape=jax.ShapeDtypeStruct((m, n), jnp.float32),
        in_specs=[
            pl.BlockSpec(memory_space=pltpu.VMEM),
            pl.BlockSpec(memory_space=pltpu.VMEM),
            pl.BlockSpec(memory_space=pltpu.VMEM),
        ],
        out_specs=pl.BlockSpec(memory_space=pltpu.VMEM),
        scratch_shapes=[
            pltpu.VMEM((2 * NUM_CHUNKS, mc, n), jnp.float32),
            pltpu.SemaphoreType.DMA((2 * NUM_CHUNKS,)),
            pltpu.SemaphoreType.DMA((2 * NUM_CHUNKS,)),
        ],
        compiler_params=pltpu.CompilerParams(collective_id=0),
    )(x, W1, W2)


# device time: 11353 ns/iter; 1.3151x vs baseline; 1.1995x over previous
import jax
import jax.numpy as jnp
from jax import lax
from jax.experimental import pallas as pl
from jax.experimental.pallas import tpu as pltpu

N_DEV = 4
NUM_CHUNKS = 4
DIRECT = 1
NB = NUM_CHUNKS - DIRECT


def kernel(x, W1, W2):
    m, _ = x.shape
    n = W2.shape[1]
    mc = m // NUM_CHUNKS

    def body(x_ref, w1_ref, w2_ref, out_ref, praw_ref, comm_ref,
             send_sems, recv_sems):
        my_pos = lax.axis_index("i")
        peers = [my_pos ^ 1, 3 - my_pos, my_pos ^ 2]

        barrier_sem = pltpu.get_barrier_semaphore()
        for nbr in peers:
            pl.semaphore_signal(
                barrier_sem, inc=1,
                device_id=(nbr,), device_id_type=pl.DeviceIdType.MESH,
            )

        def rdma_to(peer_k, slot, src):
            return pltpu.make_async_remote_copy(
                src_ref=src,
                dst_ref=comm_ref.at[slot],
                send_sem=send_sems.at[slot],
                recv_sem=recv_sems.at[slot],
                device_id=(peers[peer_k],),
                device_id_type=pl.DeviceIdType.MESH,
            )

        def bfly(stage, c):
            return rdma_to(stage, stage * NB + c,
                           out_ref.at[pl.ds(c * mc, mc), :])

        def direct(d, k):
            return rdma_to(k, 2 * NB + d * 3 + k, praw_ref.at[d])

        rdmas = {}
        for c in range(NUM_CHUNKS):
            rows = pl.ds(c * mc, mc)
            hidden = jnp.maximum(
                jnp.dot(x_ref[rows, :], w1_ref[:, :],
                        preferred_element_type=jnp.float32),
                0.0,
            )
            p = jnp.dot(hidden, w2_ref[:, :],
                        preferred_element_type=jnp.float32)
            if c == 0:
                pl.semaphore_wait(barrier_sem, len(peers))
            if c < NB:
                out_ref[rows, :] = p
                rdmas[("b", 0, c)] = bfly(0, c)
                rdmas[("b", 0, c)].start()
            else:
                d = c - NB
                praw_ref[d, :, :] = p
                for k in range(3):
                    rdmas[("d", d, k)] = direct(d, k)
                    rdmas[("d", d, k)].start()

        for c in range(NB):
            rows = pl.ds(c * mc, mc)
            rdmas[("b", 0, c)].wait()
            out_ref[rows, :] = out_ref[rows, :] + comm_ref[c, :, :]
            rdmas[("b", 1, c)] = bfly(1, c)
            rdmas[("b", 1, c)].start()

        for c in range(NB):
            rows = pl.ds(c * mc, mc)
            rdmas[("b", 1, c)].wait()
            out_ref[rows, :] = out_ref[rows, :] + comm_ref[NB + c, :, :]

        for d in range(DIRECT):
            c = NB + d
            rows = pl.ds(c * mc, mc)
            for k in range(3):
                rdmas[("d", d, k)].wait_recv()
            base = 2 * NB + d * 3
            out_ref[rows, :] = (
                praw_ref[d, :, :]
                + comm_ref[base + 0, :, :]
                + comm_ref[base + 1, :, :]
                + comm_ref[base + 2, :, :]
            )

        for d in range(DIRECT):
            for k in range(3):
                rdmas[("d", d, k)].wait_send()

    n_slots = 2 * NB + 3 * DIRECT
    return pl.pallas_call(
        body,
        out_shape=jax.ShapeDtypeStruct((m, n), jnp.float32),
        in_specs=[
            pl.BlockSpec(memory_space=pltpu.VMEM),
            pl.BlockSpec(memory_space=pltpu.VMEM),
            pl.BlockSpec(memory_space=pltpu.VMEM),
        ],
        out_specs=pl.BlockSpec(memory_space=pltpu.VMEM),
        scratch_shapes=[
            pltpu.VMEM((DIRECT, mc, n), jnp.float32),
            pltpu.VMEM((n_slots, mc, n), jnp.float32),
            pltpu.SemaphoreType.DMA((n_slots,)),
            pltpu.SemaphoreType.DMA((n_slots,)),
        ],
        compiler_params=pltpu.CompilerParams(collective_id=0),
    )(x, W1, W2)
